# baseline (speedup 1.0000x reference)
import jax
import jax.numpy as jnp
from jax.experimental import pallas as pl
from jax.experimental.pallas import tpu as pltpu

MAX_H = 128
ZROWS = 1


def _update_kernel(state_hbm, out_hbm, zeros_vmem, state_vmem, mean_vmem,
                   zsem, ssem, msem):
    zeros_vmem[...] = jnp.zeros_like(zeros_vmem)
    copies = []
    for s in range(1, MAX_H, ZROWS):
        r = min(ZROWS, MAX_H - s)
        c = pltpu.make_async_copy(
            zeros_vmem.at[pl.ds(0, r)], out_hbm.at[pl.ds(s, r)], zsem)
        c.start()
        copies.append(c)
    in_copy = pltpu.make_async_copy(state_hbm, state_vmem, ssem)
    in_copy.start()
    in_copy.wait()
    mean_vmem[...] = jnp.mean(state_vmem[...], axis=0, keepdims=True)
    m_copy = pltpu.make_async_copy(mean_vmem, out_hbm.at[pl.ds(0, 1)], msem)
    m_copy.start()
    for c in copies:
        c.wait()
    m_copy.wait()


def kernel(state):
    if state.ndim == 2:
        state = state[None, :, :]
    B, P, D = state.shape
    buf = pl.pallas_call(
        _update_kernel,
        in_specs=[pl.BlockSpec(memory_space=pltpu.MemorySpace.HBM)],
        out_specs=pl.BlockSpec(memory_space=pltpu.MemorySpace.HBM),
        out_shape=jax.ShapeDtypeStruct((MAX_H, P, D), state.dtype),
        scratch_shapes=[
            pltpu.VMEM((ZROWS, P, D), state.dtype),
            pltpu.VMEM((B, P, D), state.dtype),
            pltpu.VMEM((1, P, D), state.dtype),
            pltpu.SemaphoreType.DMA,
            pltpu.SemaphoreType.DMA,
            pltpu.SemaphoreType.DMA,
        ],
    )(state)
    return buf, jnp.asarray(1, dtype=jnp.int32)
